# baseline (speedup 1.0000x reference)
import jax
import jax.numpy as jnp
from jax.experimental import pallas as pl

_N = 100000
_D = 128
_BLK = 2000


def _body(x_ref, c_ref, o_ref):
    i = pl.program_id(0)
    diff = c_ref[...] - x_ref[...]
    m = jnp.min(jnp.sum(diff * diff, axis=1)).reshape(1, 1)

    @pl.when(i == 0)
    def _init():
        o_ref[...] = m

    @pl.when(i > 0)
    def _acc():
        o_ref[...] = jnp.minimum(o_ref[...], m)

    @pl.when(i == pl.num_programs(0) - 1)
    def _fin():
        o_ref[...] = jnp.sqrt(o_ref[...])


def kernel(x, corpus):
    out = pl.pallas_call(
        _body,
        grid=(_N // _BLK,),
        in_specs=[
            pl.BlockSpec((1, _D), lambda i: (0, 0)),
            pl.BlockSpec((_BLK, _D), lambda i: (i, 0)),
        ],
        out_specs=pl.BlockSpec((1, 1), lambda i: (0, 0)),
        out_shape=jax.ShapeDtypeStruct((1, 1), jnp.float32),
    )(x.reshape(1, _D), corpus)
    return out[0, 0]


# speedup vs baseline: 1.4584x; 1.4584x over previous
import jax
import jax.numpy as jnp
from jax.experimental import pallas as pl

_N = 100000
_D = 128
_BLK = 4000


def _body(x_ref, ones_ref, c_ref, o_ref):
    i = pl.program_id(0)
    c = c_ref[...]
    z = c * (c - 2.0 * x_ref[...])
    mm = jax.lax.dot_general(
        z, ones_ref[...], (((1,), (0,)), ((), ())),
        preferred_element_type=jnp.float32,
    )
    m = jnp.min(mm).reshape(1, 1)

    @pl.when(i == 0)
    def _init():
        o_ref[...] = m

    @pl.when(i > 0)
    def _acc():
        o_ref[...] = jnp.minimum(o_ref[...], m)

    @pl.when(i == pl.num_programs(0) - 1)
    def _fin():
        xv = x_ref[...]
        x2 = jnp.sum(xv * xv)
        o_ref[...] = jnp.sqrt(jnp.maximum(o_ref[...] + x2, 0.0))


def kernel(x, corpus):
    ones_mat = jnp.ones((_D, _D), dtype=jnp.float32)
    out = pl.pallas_call(
        _body,
        grid=(_N // _BLK,),
        in_specs=[
            pl.BlockSpec((1, _D), lambda i: (0, 0)),
            pl.BlockSpec((_D, _D), lambda i: (0, 0)),
            pl.BlockSpec((_BLK, _D), lambda i: (i, 0)),
        ],
        out_specs=pl.BlockSpec((1, 1), lambda i: (0, 0)),
        out_shape=jax.ShapeDtypeStruct((1, 1), jnp.float32),
    )(x.reshape(1, _D), ones_mat, corpus)
    return out[0, 0]
